# Initial kernel scaffold; baseline (speedup 1.0000x reference)
import jax
import jax.numpy as jnp
from jax.experimental import pallas as pl


def kernel(x, edge_index, edge_attr, atom_emb, W_A, b_A, W_B, b_B, W_C, b_C, W_U, b_U, W_V, b_V, W_o1, b_o1, W_o2, b_o2):
    raise NotImplementedError("write your pallas kernel here")


# speedup vs baseline: 2.7559x; 2.7559x over previous
import functools

import jax
import jax.numpy as jnp
from jax import lax
from jax.experimental import pallas as pl
from jax.experimental.pallas import tpu as pltpu
from jax.experimental.pallas import tpu_sc as plsc

DIM = 128
N_LAYERS = 6
CUTOFF = 6.0
N_NODES = 10000
N_EDGES = 320000

NP = 10112
CHUNK = 128
NW = 32
EDGE_CHUNKS = N_EDGES // CHUNK
NODE_CHUNKS = NP // CHUNK
ROWS_PER_TILE = NP // 16

EBLK = 4000
EGRID = N_EDGES // EBLK

@functools.cache
def _mesh():
    return plsc.VectorSubcoreMesh(core_axis_name="c", subcore_axis_name="s")


def _sigmoid(v):
    return 1.0 / (1.0 + jnp.exp(-v))


def _edge_init_body(ea_ref, e_ref):
    r = ea_ref[0, 0, :]
    xcol = r[:, None] + 1e-6
    n = (lax.broadcasted_iota(jnp.int32, (1, DIM), 1) + 1).astype(jnp.float32)
    c = CUTOFF
    e_ref[...] = jnp.sqrt(2.0 / c) * jnp.sin(n * (jnp.pi / c) * xcol) / xcol


_edge_init = pl.pallas_call(
    _edge_init_body,
    grid=(EGRID,),
    in_specs=[pl.BlockSpec((1, 1, EBLK), lambda i: (i, 0, 0))],
    out_specs=pl.BlockSpec((EBLK, DIM), lambda i: (i, 0)),
    out_shape=jax.ShapeDtypeStruct((N_EDGES, DIM), jnp.float32),
)


def _node_proj_body(h_ref, wav_ref, bav_ref, wb_ref, bb_ref, hav_ref, hb_ref):
    h = h_ref[...]
    hav_ref[...] = jnp.dot(h, wav_ref[...],
                           preferred_element_type=jnp.float32) + bav_ref[...]
    hb_ref[...] = jnp.dot(h, wb_ref[...],
                          preferred_element_type=jnp.float32) + bb_ref[...]


_node_proj = pl.pallas_call(
    _node_proj_body,
    out_shape=(jax.ShapeDtypeStruct((NP, 2 * DIM), jnp.float32),
               jax.ShapeDtypeStruct((NP, DIM), jnp.float32)),
)


def _edge_compute_body(e_ref, gav_ref, gb_ref, wc_ref, bc_ref,
                       enew_ref, msg_ref, sig_ref):
    e = e_ref[...]
    gav = gav_ref[...]
    m = (gav[:, :DIM] + gb_ref[...]
         + jnp.dot(e, wc_ref[...], preferred_element_type=jnp.float32)
         + bc_ref[...])
    sg = _sigmoid(m)
    sig_ref[...] = sg
    msg_ref[...] = sg * gav[:, DIM:]
    enew_ref[...] = e + m * sg


_edge_compute = pl.pallas_call(
    _edge_compute_body,
    grid=(EGRID,),
    in_specs=[
        pl.BlockSpec((EBLK, DIM), lambda i: (i, 0)),
        pl.BlockSpec((EBLK, 2 * DIM), lambda i: (i, 0)),
        pl.BlockSpec((EBLK, DIM), lambda i: (i, 0)),
        pl.BlockSpec((DIM, DIM), lambda i: (0, 0)),
        pl.BlockSpec((1, DIM), lambda i: (0, 0)),
    ],
    out_specs=[
        pl.BlockSpec((EBLK, DIM), lambda i: (i, 0)),
        pl.BlockSpec((EBLK, DIM), lambda i: (i, 0)),
        pl.BlockSpec((EBLK, DIM), lambda i: (i, 0)),
    ],
    out_shape=(jax.ShapeDtypeStruct((N_EDGES, DIM), jnp.float32),
               jax.ShapeDtypeStruct((N_EDGES, DIM), jnp.float32),
               jax.ShapeDtypeStruct((N_EDGES, DIM), jnp.float32)),
)


def _node_update_body(h_ref, np_ref, dp_ref, wu_ref, bu_ref, hn_ref):
    h = h_ref[...]
    num = np_ref[0] + np_ref[1]
    den = dp_ref[0] + dp_ref[1] + 1e-6
    u = (jnp.dot(h, wu_ref[...], preferred_element_type=jnp.float32)
         + bu_ref[...] + num / den)
    hn_ref[...] = h + u * _sigmoid(u)


_node_update = pl.pallas_call(
    _node_update_body,
    out_shape=jax.ShapeDtypeStruct((NP, DIM), jnp.float32),
)


def _out_head_body(h_ref, w1_ref, b1_ref, w2_ref, b2_ref, o_ref):
    t = (jnp.dot(h_ref[...], w1_ref[...], preferred_element_type=jnp.float32)
         + b1_ref[...])
    t = t * _sigmoid(t)
    z = jnp.sum(t * w2_ref[...], axis=1, keepdims=True) + b2_ref[...]
    o_ref[...] = _sigmoid(z)


_out_head = pl.pallas_call(
    _out_head_body,
    out_shape=jax.ShapeDtypeStruct((NP, 1), jnp.float32),
)


@functools.cache
def _make_gather(n_rows_out, depth):
    n_chunks = n_rows_out // CHUNK
    iters = (n_chunks + NW - 1) // NW

    @functools.partial(
        pl.kernel, mesh=_mesh(),
        out_type=jax.ShapeDtypeStruct((n_rows_out, depth), jnp.float32),
        scratch_types=[
            pltpu.VMEM((CHUNK,), jnp.int32),
            pltpu.VMEM((CHUNK, depth), jnp.float32),
            pltpu.SemaphoreType.DMA,
        ],
    )
    def gather(table_hbm, idx_hbm, out_hbm, idx_v, rows_v, sem):
        w = lax.axis_index("s") * 2 + lax.axis_index("c")

        def body(i, carry):
            g = i * NW + w

            @pl.when(g < n_chunks)
            def _():
                base = g * CHUNK
                pltpu.sync_copy(idx_hbm.at[pl.ds(base, CHUNK)], idx_v)
                pltpu.async_copy(table_hbm.at[idx_v], rows_v, sem).wait()
                pltpu.sync_copy(rows_v, out_hbm.at[pl.ds(base, CHUNK), :])

            return carry

        lax.fori_loop(0, iters, body, 0)

    return gather


@functools.cache
def _sc_scatter_kernel():
    iters = (EDGE_CHUNKS + NW - 1) // NW

    @functools.partial(
        pl.kernel, mesh=_mesh(),
        out_type=(jax.ShapeDtypeStruct((2, NP, DIM), jnp.float32),
                  jax.ShapeDtypeStruct((2, NP, DIM), jnp.float32)),
        scratch_types=[
            pltpu.VMEM_SHARED((NP, DIM), jnp.float32),
            pltpu.VMEM((CHUNK,), jnp.int32),
            pltpu.VMEM((CHUNK, DIM), jnp.float32),
        ],
    )
    def scatter(msg_hbm, sig_hbm, dst_hbm, zeros_hbm,
                num_hbm, den_hbm, accum, idx_v, rows_v):
        c = lax.axis_index("c")
        s = lax.axis_index("s")
        w = s * 2 + c
        rbase = s * ROWS_PER_TILE
        rows = pl.ds(rbase, ROWS_PER_TILE)

        def one_phase(src_hbm, out_hbm):
            pltpu.sync_copy(zeros_hbm.at[rows], accum.at[rows])
            plsc.subcore_barrier()

            def body(i, carry):
                g = i * NW + w

                @pl.when(g < EDGE_CHUNKS)
                def _():
                    base = g * CHUNK
                    pltpu.sync_copy(dst_hbm.at[pl.ds(base, CHUNK)], idx_v)
                    pltpu.sync_copy(src_hbm.at[pl.ds(base, CHUNK), :], rows_v)
                    pltpu.sync_copy(rows_v, accum.at[idx_v], add=True)

                return carry

            lax.fori_loop(0, iters, body, 0)
            plsc.subcore_barrier()
            pltpu.sync_copy(accum.at[rows], out_hbm.at[c, rows])
            plsc.subcore_barrier()

        one_phase(msg_hbm, num_hbm)
        one_phase(sig_hbm, den_hbm)

    return scatter


def kernel(x, edge_index, edge_attr, atom_emb,
           W_A, b_A, W_B, b_B, W_C, b_C, W_U, b_U, W_V, b_V,
           W_o1, b_o1, W_o2, b_o2):
    src = edge_index[0].astype(jnp.int32)
    dst = edge_index[1].astype(jnp.int32)
    x_p = jnp.concatenate(
        [x.astype(jnp.int32), jnp.zeros((NP - N_NODES,), jnp.int32)])
    ea3 = edge_attr.reshape(EGRID, 1, EBLK)

    W_AV = jnp.concatenate([W_A, W_V], axis=2)
    b_AV = jnp.concatenate([b_A, b_V], axis=1)
    zeros_np = jnp.zeros((NP, DIM), jnp.float32)

    gather_emb = _make_gather(NP, DIM)
    gather_av = _make_gather(N_EDGES, 2 * DIM)
    gather_b = _make_gather(N_EDGES, DIM)
    sc_scatter = _sc_scatter_kernel()

    h = gather_emb(atom_emb, x_p)
    e = _edge_init(ea3)

    for i in range(N_LAYERS):
        hav, hb = _node_proj(h, W_AV[i], b_AV[i].reshape(1, 2 * DIM),
                             W_B[i], b_B[i].reshape(1, DIM))
        gav = gather_av(hav, src)
        gb = gather_b(hb, dst)
        e, msg, sig = _edge_compute(e, gav, gb, W_C[i],
                                    b_C[i].reshape(1, DIM))
        nparts, dparts = sc_scatter(msg, sig, dst, zeros_np)
        h = _node_update(h, nparts, dparts, W_U[i], b_U[i].reshape(1, DIM))

    out = _out_head(h, W_o1, b_o1.reshape(1, DIM),
                    W_o2[:, 0].reshape(1, DIM), b_o2.reshape(1, 1))
    return out[:N_NODES]


# speedup vs baseline: 3.8050x; 1.3807x over previous
import functools

import jax
import jax.numpy as jnp
from jax import lax
from jax.experimental import pallas as pl
from jax.experimental.pallas import tpu as pltpu
from jax.experimental.pallas import tpu_sc as plsc

DIM = 128
N_LAYERS = 6
CUTOFF = 6.0
N_NODES = 10000
N_EDGES = 320000

NP = 10112
CHUNK = 128
NW = 32
EDGE_CHUNKS = N_EDGES // CHUNK
NODE_CHUNKS = NP // CHUNK
ROWS_PER_TILE = NP // 16

EBLK = 4000
EGRID = N_EDGES // EBLK

@functools.cache
def _mesh():
    return plsc.VectorSubcoreMesh(core_axis_name="c", subcore_axis_name="s")


def _sigmoid(v):
    return 1.0 / (1.0 + jnp.exp(-v))


def _edge_init_body(ea_ref, e_ref):
    r = ea_ref[0, 0, :]
    xcol = r[:, None] + 1e-6
    n = (lax.broadcasted_iota(jnp.int32, (1, DIM), 1) + 1).astype(jnp.float32)
    c = CUTOFF
    e_ref[...] = jnp.sqrt(2.0 / c) * jnp.sin(n * (jnp.pi / c) * xcol) / xcol


_edge_init = pl.pallas_call(
    _edge_init_body,
    grid=(EGRID,),
    in_specs=[pl.BlockSpec((1, 1, EBLK), lambda i: (i, 0, 0))],
    out_specs=pl.BlockSpec((EBLK, DIM), lambda i: (i, 0)),
    out_shape=jax.ShapeDtypeStruct((N_EDGES, DIM), jnp.float32),
)


def _node_proj_body(h_ref, wav_ref, bav_ref, wb_ref, bb_ref, hav_ref, hb_ref):
    h = h_ref[...]
    hav_ref[...] = jnp.dot(h, wav_ref[...],
                           preferred_element_type=jnp.float32) + bav_ref[...]
    hb_ref[...] = jnp.dot(h, wb_ref[...],
                          preferred_element_type=jnp.float32) + bb_ref[...]


_node_proj = pl.pallas_call(
    _node_proj_body,
    out_shape=(jax.ShapeDtypeStruct((NP, 2 * DIM), jnp.float32),
               jax.ShapeDtypeStruct((NP, DIM), jnp.float32)),
)


def _edge_compute_body(e_ref, gav_ref, gb_ref, wc_ref, bc_ref,
                       enew_ref, msg_ref, sig_ref):
    e = e_ref[...]
    gav = gav_ref[...]
    m = (gav[:, :DIM] + gb_ref[...]
         + jnp.dot(e, wc_ref[...], preferred_element_type=jnp.float32)
         + bc_ref[...])
    sg = _sigmoid(m)
    sig_ref[...] = sg
    msg_ref[...] = sg * gav[:, DIM:]
    enew_ref[...] = e + m * sg


_edge_compute = pl.pallas_call(
    _edge_compute_body,
    grid=(EGRID,),
    in_specs=[
        pl.BlockSpec((EBLK, DIM), lambda i: (i, 0)),
        pl.BlockSpec((EBLK, 2 * DIM), lambda i: (i, 0)),
        pl.BlockSpec((EBLK, DIM), lambda i: (i, 0)),
        pl.BlockSpec((DIM, DIM), lambda i: (0, 0)),
        pl.BlockSpec((1, DIM), lambda i: (0, 0)),
    ],
    out_specs=[
        pl.BlockSpec((EBLK, DIM), lambda i: (i, 0)),
        pl.BlockSpec((EBLK, DIM), lambda i: (i, 0)),
        pl.BlockSpec((EBLK, DIM), lambda i: (i, 0)),
    ],
    out_shape=(jax.ShapeDtypeStruct((N_EDGES, DIM), jnp.float32),
               jax.ShapeDtypeStruct((N_EDGES, DIM), jnp.float32),
               jax.ShapeDtypeStruct((N_EDGES, DIM), jnp.float32)),
)


def _node_update_body(h_ref, np_ref, dp_ref, wu_ref, bu_ref, hn_ref):
    h = h_ref[...]
    num = np_ref[0] + np_ref[1]
    den = dp_ref[0] + dp_ref[1] + 1e-6
    u = (jnp.dot(h, wu_ref[...], preferred_element_type=jnp.float32)
         + bu_ref[...] + num / den)
    hn_ref[...] = h + u * _sigmoid(u)


_node_update = pl.pallas_call(
    _node_update_body,
    out_shape=jax.ShapeDtypeStruct((NP, DIM), jnp.float32),
)


def _out_head_body(h_ref, w1_ref, b1_ref, w2_ref, b2_ref, o_ref):
    t = (jnp.dot(h_ref[...], w1_ref[...], preferred_element_type=jnp.float32)
         + b1_ref[...])
    t = t * _sigmoid(t)
    z = jnp.sum(t * w2_ref[...], axis=1, keepdims=True) + b2_ref[...]
    o_ref[...] = _sigmoid(z)


_out_head = pl.pallas_call(
    _out_head_body,
    out_shape=jax.ShapeDtypeStruct((NP, 1), jnp.float32),
)


@functools.cache
def _make_gather(n_rows_out, depth):
    n_chunks = n_rows_out // CHUNK
    iters = (n_chunks + NW - 1) // NW

    @functools.partial(
        pl.kernel, mesh=_mesh(),
        out_type=jax.ShapeDtypeStruct((n_rows_out, depth), jnp.float32),
        scratch_types=[
            pltpu.VMEM((CHUNK,), jnp.int32),
            pltpu.VMEM((CHUNK, depth), jnp.float32),
            pltpu.SemaphoreType.DMA,
        ],
    )
    def gather(table_hbm, idx_hbm, out_hbm, idx_v, rows_v, sem):
        w = lax.axis_index("s") * 2 + lax.axis_index("c")

        def body(i, carry):
            g = i * NW + w

            @pl.when(g < n_chunks)
            def _():
                base = g * CHUNK
                pltpu.sync_copy(idx_hbm.at[pl.ds(base, CHUNK)], idx_v)
                pltpu.async_copy(table_hbm.at[idx_v], rows_v, sem).wait()
                pltpu.sync_copy(rows_v, out_hbm.at[pl.ds(base, CHUNK), :])

            return carry

        lax.fori_loop(0, iters, body, 0)

    return gather


MAX_CH = 80
N_CHUNKS_PAD = MAX_CH * NW


def _tile_range(w):
    start = pl.multiple_of(MAX_CH * w, 8)
    nch = jnp.minimum(EDGE_CHUNKS - MAX_CH * w, MAX_CH)
    return start, nch


@functools.cache
def _make_edge_gather():
    @functools.partial(
        pl.kernel, mesh=_mesh(),
        out_type=(jax.ShapeDtypeStruct((N_EDGES, 2 * DIM), jnp.float32),
                  jax.ShapeDtypeStruct((N_EDGES, DIM), jnp.float32)),
        scratch_types=[
            pltpu.VMEM((MAX_CH, CHUNK), jnp.int32),
            pltpu.VMEM((MAX_CH, CHUNK), jnp.int32),
            pltpu.VMEM((CHUNK, 2 * DIM), jnp.float32),
            pltpu.VMEM((CHUNK, 2 * DIM), jnp.float32),
            pltpu.VMEM((CHUNK, DIM), jnp.float32),
            pltpu.VMEM((CHUNK, DIM), jnp.float32),
            pltpu.SemaphoreType.DMA,
            pltpu.SemaphoreType.DMA,
            pltpu.SemaphoreType.DMA,
            pltpu.SemaphoreType.DMA,
            pltpu.SemaphoreType.DMA,
        ],
    )
    def gather(tab_av, tab_b, src2, dst2, gav, gb,
               sidx, didx, rav0, rav1, rb0, rb1, lsem, g0, g1, w0, w1):
        w = lax.axis_index("s") * 2 + lax.axis_index("c")
        start, nch = _tile_range(w)
        RAV, RB, GS, WS = (rav0, rav1), (rb0, rb1), (g0, g1), (w0, w1)

        pltpu.async_copy(src2.at[pl.ds(start, MAX_CH), :], sidx, lsem)
        pltpu.async_copy(dst2.at[pl.ds(start, MAX_CH), :], didx, lsem)
        pltpu.make_async_copy(src2.at[pl.ds(start, MAX_CH), :], sidx,
                              lsem).wait()
        pltpu.make_async_copy(dst2.at[pl.ds(start, MAX_CH), :], didx,
                              lsem).wait()

        def body(i2, carry):
            for u in (0, 1):
                i = i2 * 2 + u
                base = pl.multiple_of((start + i) * CHUNK, CHUNK)
                out_av = gav.at[pl.ds(base, CHUNK), :]
                out_b = gb.at[pl.ds(base, CHUNK), :]

                @pl.when((i >= 2) & (i - 2 < nch))
                def _():
                    pltpu.make_async_copy(RAV[u], out_av, WS[u]).wait()
                    pltpu.make_async_copy(RB[u], out_b, WS[u]).wait()

                @pl.when(i < nch)
                def _():
                    pltpu.async_copy(tab_av.at[sidx.at[i]], RAV[u], GS[u])
                    pltpu.async_copy(tab_b.at[didx.at[i]], RB[u], GS[u])
                    pltpu.make_async_copy(tab_av.at[sidx.at[i]], RAV[u],
                                          GS[u]).wait()
                    pltpu.make_async_copy(tab_b.at[didx.at[i]], RB[u],
                                          GS[u]).wait()
                    pltpu.async_copy(RAV[u], out_av, WS[u])
                    pltpu.async_copy(RB[u], out_b, WS[u])

            return carry

        lax.fori_loop(0, (MAX_CH + 3) // 2, body, 0)

    return gather


@functools.cache
def _sc_scatter_kernel():
    @functools.partial(
        pl.kernel, mesh=_mesh(),
        out_type=(jax.ShapeDtypeStruct((2, NP, DIM), jnp.float32),
                  jax.ShapeDtypeStruct((2, NP, DIM), jnp.float32)),
        scratch_types=[
            pltpu.VMEM_SHARED((NP, DIM), jnp.float32),
            pltpu.VMEM((MAX_CH, CHUNK), jnp.int32),
            pltpu.VMEM((CHUNK, DIM), jnp.float32),
            pltpu.VMEM((CHUNK, DIM), jnp.float32),
            pltpu.SemaphoreType.DMA,
            pltpu.SemaphoreType.DMA,
            pltpu.SemaphoreType.DMA,
            pltpu.SemaphoreType.DMA,
        ],
    )
    def scatter(msg_hbm, sig_hbm, dst2, zeros_hbm,
                num_hbm, den_hbm, accum, didx, r0, r1, l0, l1, s0, s1):
        c = lax.axis_index("c")
        s = lax.axis_index("s")
        w = s * 2 + c
        start, nch = _tile_range(w)
        rows = pl.ds(pl.multiple_of(s * ROWS_PER_TILE, 8), ROWS_PER_TILE)
        R, L, S = (r0, r1), (l0, l1), (s0, s1)

        pltpu.sync_copy(dst2.at[pl.ds(start, MAX_CH), :], didx)

        def one_phase(src_hbm, out_hbm):
            pltpu.sync_copy(zeros_hbm.at[rows], accum.at[rows])
            plsc.subcore_barrier()

            @pl.when(0 < nch)
            def _():
                pltpu.async_copy(
                    src_hbm.at[pl.ds(pl.multiple_of(start * CHUNK, CHUNK),
                                     CHUNK), :], R[0], L[0])

            def body(i2, carry):
                for u in (0, 1):
                    i = i2 * 2 + u
                    cur = src_hbm.at[
                        pl.ds(pl.multiple_of((start + i) * CHUNK, CHUNK),
                              CHUNK), :]

                    @pl.when((i >= 1) & (i - 1 < nch))
                    def _():
                        pltpu.make_async_copy(
                            R[1 - u], accum.at[didx.at[i]], S[1 - u]).wait()

                    @pl.when(i + 1 < nch)
                    def _():
                        pltpu.async_copy(
                            src_hbm.at[
                                pl.ds(pl.multiple_of((start + i + 1) * CHUNK,
                                                     CHUNK), CHUNK), :],
                            R[1 - u], L[1 - u])

                    @pl.when(i < nch)
                    def _():
                        pltpu.make_async_copy(cur, R[u], L[u]).wait()
                        pltpu.async_copy(R[u], accum.at[didx.at[i]], S[u],
                                         add=True)

                return carry

            lax.fori_loop(0, (MAX_CH + 3) // 2, body, 0)
            plsc.subcore_barrier()
            pltpu.sync_copy(accum.at[rows], out_hbm.at[c, rows])
            plsc.subcore_barrier()

        one_phase(msg_hbm, num_hbm)
        one_phase(sig_hbm, den_hbm)

    return scatter


def kernel(x, edge_index, edge_attr, atom_emb,
           W_A, b_A, W_B, b_B, W_C, b_C, W_U, b_U, W_V, b_V,
           W_o1, b_o1, W_o2, b_o2):
    pad = N_CHUNKS_PAD * CHUNK - N_EDGES
    src2 = jnp.concatenate(
        [edge_index[0].astype(jnp.int32), jnp.zeros((pad,), jnp.int32)]
    ).reshape(N_CHUNKS_PAD, CHUNK)
    dst2 = jnp.concatenate(
        [edge_index[1].astype(jnp.int32), jnp.zeros((pad,), jnp.int32)]
    ).reshape(N_CHUNKS_PAD, CHUNK)
    x_p = jnp.concatenate(
        [x.astype(jnp.int32), jnp.zeros((NP - N_NODES,), jnp.int32)])
    ea3 = edge_attr.reshape(EGRID, 1, EBLK)

    W_AV = jnp.concatenate([W_A, W_V], axis=2)
    b_AV = jnp.concatenate([b_A, b_V], axis=1)
    zeros_np = jnp.zeros((NP, DIM), jnp.float32)

    gather_emb = _make_gather(NP, DIM)
    edge_gather = _make_edge_gather()
    sc_scatter = _sc_scatter_kernel()

    h = gather_emb(atom_emb, x_p)
    e = _edge_init(ea3)

    for i in range(N_LAYERS):
        hav, hb = _node_proj(h, W_AV[i], b_AV[i].reshape(1, 2 * DIM),
                             W_B[i], b_B[i].reshape(1, DIM))
        gav, gb = edge_gather(hav, hb, src2, dst2)
        e, msg, sig = _edge_compute(e, gav, gb, W_C[i],
                                    b_C[i].reshape(1, DIM))
        nparts, dparts = sc_scatter(msg, sig, dst2, zeros_np)
        h = _node_update(h, nparts, dparts, W_U[i], b_U[i].reshape(1, DIM))

    out = _out_head(h, W_o1, b_o1.reshape(1, DIM),
                    W_o2[:, 0].reshape(1, DIM), b_o2.reshape(1, 1))
    return out[:N_NODES]


# speedup vs baseline: 3.8910x; 1.0226x over previous
import functools

import jax
import jax.numpy as jnp
from jax import lax
from jax.experimental import pallas as pl
from jax.experimental.pallas import tpu as pltpu
from jax.experimental.pallas import tpu_sc as plsc

DIM = 128
N_LAYERS = 6
CUTOFF = 6.0
N_NODES = 10000
N_EDGES = 320000

NP = 10112
CHUNK = 128
NW = 32
EDGE_CHUNKS = N_EDGES // CHUNK
NODE_CHUNKS = NP // CHUNK
ROWS_PER_TILE = NP // 16

EBLK = 4000
EGRID = N_EDGES // EBLK

@functools.cache
def _mesh():
    return plsc.VectorSubcoreMesh(core_axis_name="c", subcore_axis_name="s")


def _sigmoid(v):
    return 1.0 / (1.0 + jnp.exp(-v))


def _edge_init_body(ea_ref, e_ref):
    r = ea_ref[0, 0, :]
    xcol = r[:, None] + 1e-6
    n = (lax.broadcasted_iota(jnp.int32, (1, DIM), 1) + 1).astype(jnp.float32)
    c = CUTOFF
    e_ref[...] = jnp.sqrt(2.0 / c) * jnp.sin(n * (jnp.pi / c) * xcol) / xcol


_edge_init = pl.pallas_call(
    _edge_init_body,
    grid=(EGRID,),
    in_specs=[pl.BlockSpec((1, 1, EBLK), lambda i: (i, 0, 0))],
    out_specs=pl.BlockSpec((EBLK, DIM), lambda i: (i, 0)),
    out_shape=jax.ShapeDtypeStruct((N_EDGES, DIM), jnp.float32),
)


def _node_proj_body(h_ref, wav_ref, bav_ref, wb_ref, bb_ref, hav_ref, hb_ref):
    h = h_ref[...]
    hav_ref[...] = jnp.dot(h, wav_ref[...],
                           preferred_element_type=jnp.float32) + bav_ref[...]
    hb_ref[...] = jnp.dot(h, wb_ref[...],
                          preferred_element_type=jnp.float32) + bb_ref[...]


_node_proj = pl.pallas_call(
    _node_proj_body,
    out_shape=(jax.ShapeDtypeStruct((NP, 2 * DIM), jnp.float32),
               jax.ShapeDtypeStruct((NP, DIM), jnp.float32)),
)


def _edge_compute_body(e_ref, gav_ref, gb_ref, wc_ref, bc_ref,
                       enew_ref, msg_ref, sig_ref):
    e = e_ref[...]
    gav = gav_ref[...]
    m = (gav[:, :DIM] + gb_ref[...]
         + jnp.dot(e, wc_ref[...], preferred_element_type=jnp.float32)
         + bc_ref[...])
    sg = _sigmoid(m)
    sig_ref[...] = sg
    msg_ref[...] = sg * gav[:, DIM:]
    enew_ref[...] = e + m * sg


_edge_compute = pl.pallas_call(
    _edge_compute_body,
    grid=(EGRID,),
    in_specs=[
        pl.BlockSpec((EBLK, DIM), lambda i: (i, 0)),
        pl.BlockSpec((EBLK, 2 * DIM), lambda i: (i, 0)),
        pl.BlockSpec((EBLK, DIM), lambda i: (i, 0)),
        pl.BlockSpec((DIM, DIM), lambda i: (0, 0)),
        pl.BlockSpec((1, DIM), lambda i: (0, 0)),
    ],
    out_specs=[
        pl.BlockSpec((EBLK, DIM), lambda i: (i, 0)),
        pl.BlockSpec((EBLK, DIM), lambda i: (i, 0)),
        pl.BlockSpec((EBLK, DIM), lambda i: (i, 0)),
    ],
    out_shape=(jax.ShapeDtypeStruct((N_EDGES, DIM), jnp.float32),
               jax.ShapeDtypeStruct((N_EDGES, DIM), jnp.float32),
               jax.ShapeDtypeStruct((N_EDGES, DIM), jnp.float32)),
)


def _node_update_body(h_ref, np_ref, dp_ref, wu_ref, bu_ref, hn_ref):
    h = h_ref[...]
    u = (jnp.dot(h, wu_ref[...], preferred_element_type=jnp.float32)
         + bu_ref[...] + np_ref[...] / (dp_ref[...] + 1e-6))
    hn_ref[...] = h + u * _sigmoid(u)


_node_update = pl.pallas_call(
    _node_update_body,
    out_shape=jax.ShapeDtypeStruct((NP, DIM), jnp.float32),
)


def _node_update_proj_body(h_ref, np_ref, dp_ref, wu_ref, bu_ref,
                           wav_ref, bav_ref, wb_ref, bb_ref,
                           hn_ref, hav_ref, hb_ref):
    h = h_ref[...]
    u = (jnp.dot(h, wu_ref[...], preferred_element_type=jnp.float32)
         + bu_ref[...] + np_ref[...] / (dp_ref[...] + 1e-6))
    hn = h + u * _sigmoid(u)
    hn_ref[...] = hn
    hav_ref[...] = jnp.dot(hn, wav_ref[...],
                           preferred_element_type=jnp.float32) + bav_ref[...]
    hb_ref[...] = jnp.dot(hn, wb_ref[...],
                          preferred_element_type=jnp.float32) + bb_ref[...]


_node_update_proj = pl.pallas_call(
    _node_update_proj_body,
    out_shape=(jax.ShapeDtypeStruct((NP, DIM), jnp.float32),
               jax.ShapeDtypeStruct((NP, 2 * DIM), jnp.float32),
               jax.ShapeDtypeStruct((NP, DIM), jnp.float32)),
)


def _out_head_body(h_ref, w1_ref, b1_ref, w2_ref, b2_ref, o_ref):
    t = (jnp.dot(h_ref[...], w1_ref[...], preferred_element_type=jnp.float32)
         + b1_ref[...])
    t = t * _sigmoid(t)
    z = jnp.sum(t * w2_ref[...], axis=1, keepdims=True) + b2_ref[...]
    o_ref[...] = _sigmoid(z)


_out_head = pl.pallas_call(
    _out_head_body,
    out_shape=jax.ShapeDtypeStruct((NP, 1), jnp.float32),
)


@functools.cache
def _make_gather(n_rows_out, depth):
    n_chunks = n_rows_out // CHUNK
    iters = (n_chunks + NW - 1) // NW

    @functools.partial(
        pl.kernel, mesh=_mesh(),
        out_type=jax.ShapeDtypeStruct((n_rows_out, depth), jnp.float32),
        scratch_types=[
            pltpu.VMEM((CHUNK,), jnp.int32),
            pltpu.VMEM((CHUNK, depth), jnp.float32),
            pltpu.SemaphoreType.DMA,
        ],
    )
    def gather(table_hbm, idx_hbm, out_hbm, idx_v, rows_v, sem):
        w = lax.axis_index("s") * 2 + lax.axis_index("c")

        def body(i, carry):
            g = i * NW + w

            @pl.when(g < n_chunks)
            def _():
                base = g * CHUNK
                pltpu.sync_copy(idx_hbm.at[pl.ds(base, CHUNK)], idx_v)
                pltpu.async_copy(table_hbm.at[idx_v], rows_v, sem).wait()
                pltpu.sync_copy(rows_v, out_hbm.at[pl.ds(base, CHUNK), :])

            return carry

        lax.fori_loop(0, iters, body, 0)

    return gather


MAX_CH = 80
N_CHUNKS_PAD = MAX_CH * NW


def _tile_range(w):
    start = pl.multiple_of(MAX_CH * w, 8)
    nch = jnp.minimum(EDGE_CHUNKS - MAX_CH * w, MAX_CH)
    return start, nch


@functools.cache
def _make_edge_gather():
    @functools.partial(
        pl.kernel, mesh=_mesh(),
        out_type=(jax.ShapeDtypeStruct((N_EDGES, 2 * DIM), jnp.float32),
                  jax.ShapeDtypeStruct((N_EDGES, DIM), jnp.float32)),
        scratch_types=[
            pltpu.VMEM((MAX_CH, CHUNK), jnp.int32),
            pltpu.VMEM((MAX_CH, CHUNK), jnp.int32),
            pltpu.VMEM((CHUNK, 2 * DIM), jnp.float32),
            pltpu.VMEM((CHUNK, 2 * DIM), jnp.float32),
            pltpu.VMEM((CHUNK, DIM), jnp.float32),
            pltpu.VMEM((CHUNK, DIM), jnp.float32),
            pltpu.SemaphoreType.DMA,
            pltpu.SemaphoreType.DMA,
            pltpu.SemaphoreType.DMA,
            pltpu.SemaphoreType.DMA,
            pltpu.SemaphoreType.DMA,
        ],
    )
    def gather(tab_av, tab_b, src2, dst2, gav, gb,
               sidx, didx, rav0, rav1, rb0, rb1, lsem, g0, g1, w0, w1):
        w = lax.axis_index("s") * 2 + lax.axis_index("c")
        start, nch = _tile_range(w)
        RAV, RB, GS, WS = (rav0, rav1), (rb0, rb1), (g0, g1), (w0, w1)

        pltpu.async_copy(src2.at[pl.ds(start, MAX_CH), :], sidx, lsem)
        pltpu.async_copy(dst2.at[pl.ds(start, MAX_CH), :], didx, lsem)
        pltpu.make_async_copy(src2.at[pl.ds(start, MAX_CH), :], sidx,
                              lsem).wait()
        pltpu.make_async_copy(dst2.at[pl.ds(start, MAX_CH), :], didx,
                              lsem).wait()

        def body(i2, carry):
            for u in (0, 1):
                i = i2 * 2 + u
                base = pl.multiple_of((start + i) * CHUNK, CHUNK)
                out_av = gav.at[pl.ds(base, CHUNK), :]
                out_b = gb.at[pl.ds(base, CHUNK), :]

                @pl.when((i >= 2) & (i - 2 < nch))
                def _():
                    pltpu.make_async_copy(RAV[u], out_av, WS[u]).wait()
                    pltpu.make_async_copy(RB[u], out_b, WS[u]).wait()

                @pl.when(i < nch)
                def _():
                    pltpu.async_copy(tab_av.at[sidx.at[i]], RAV[u], GS[u])
                    pltpu.async_copy(tab_b.at[didx.at[i]], RB[u], GS[u])
                    pltpu.make_async_copy(tab_av.at[sidx.at[i]], RAV[u],
                                          GS[u]).wait()
                    pltpu.make_async_copy(tab_b.at[didx.at[i]], RB[u],
                                          GS[u]).wait()
                    pltpu.async_copy(RAV[u], out_av, WS[u])
                    pltpu.async_copy(RB[u], out_b, WS[u])

            return carry

        lax.fori_loop(0, (MAX_CH + 3) // 2, body, 0)

    return gather


@functools.cache
def _sc_scatter_kernel():
    TCH = 160

    @functools.partial(
        pl.kernel, mesh=_mesh(),
        out_type=(jax.ShapeDtypeStruct((NP, DIM), jnp.float32),
                  jax.ShapeDtypeStruct((NP, DIM), jnp.float32)),
        scratch_types=[
            pltpu.VMEM_SHARED((NP, DIM), jnp.float32),
            pltpu.VMEM((CHUNK,), jnp.int32),
            pltpu.VMEM((CHUNK,), jnp.int32),
            pltpu.VMEM((CHUNK, DIM), jnp.float32),
            pltpu.VMEM((CHUNK, DIM), jnp.float32),
            pltpu.SemaphoreType.DMA,
            pltpu.SemaphoreType.DMA,
            pltpu.SemaphoreType.DMA,
        ],
    )
    def scatter(msg_hbm, sig_hbm, dstf, zeros_hbm,
                num_hbm, den_hbm, accum, i0, i1, r0, r1, l0, l1, ssem):
        c = lax.axis_index("c")
        s = lax.axis_index("s")
        start = pl.multiple_of(s * TCH, 8)
        nch = jnp.minimum(EDGE_CHUNKS - s * TCH, TCH)
        rows = pl.ds(pl.multiple_of(s * ROWS_PER_TILE, 8), ROWS_PER_TILE)
        R, I, L = (r0, r1), (i0, i1), (l0, l1)

        pltpu.sync_copy(zeros_hbm.at[rows], accum.at[rows])
        plsc.subcore_barrier()

        def issue_loads(src_hbm, i, slot):
            base = pl.multiple_of((start + i) * CHUNK, CHUNK)
            pltpu.async_copy(src_hbm.at[pl.ds(base, CHUNK), :], R[slot],
                             L[slot])
            pltpu.async_copy(dstf.at[pl.ds(base, CHUNK)], I[slot], L[slot])

        def wait_loads(src_hbm, i, slot):
            base = pl.multiple_of((start + i) * CHUNK, CHUNK)
            pltpu.make_async_copy(src_hbm.at[pl.ds(base, CHUNK), :], R[slot],
                                  L[slot]).wait()
            pltpu.make_async_copy(dstf.at[pl.ds(base, CHUNK)], I[slot],
                                  L[slot]).wait()

        def one_phase(src_hbm, out_hbm):
            @pl.when(0 < nch)
            def _():
                issue_loads(src_hbm, 0, 0)

            def body(i2, carry):
                for u in (0, 1):
                    i = i2 * 2 + u

                    @pl.when((i >= 1) & (i - 1 < nch))
                    def _():
                        pltpu.make_async_copy(
                            R[1 - u], accum.at[I[1 - u]], ssem).wait()

                    @pl.when(i + 1 < nch)
                    def _():
                        issue_loads(src_hbm, i + 1, 1 - u)

                    @pl.when(i < nch)
                    def _():
                        wait_loads(src_hbm, i, u)
                        pltpu.async_copy(R[u], accum.at[I[u]], ssem,
                                         add=True)

                return carry

            lax.fori_loop(0, (TCH + 3) // 2, body, 0)
            plsc.subcore_barrier()
            pltpu.sync_copy(accum.at[rows], out_hbm.at[rows])

        @pl.when(c == 0)
        def _():
            one_phase(msg_hbm, num_hbm)

        @pl.when(c == 1)
        def _():
            one_phase(sig_hbm, den_hbm)

    return scatter


def kernel(x, edge_index, edge_attr, atom_emb,
           W_A, b_A, W_B, b_B, W_C, b_C, W_U, b_U, W_V, b_V,
           W_o1, b_o1, W_o2, b_o2):
    pad = N_CHUNKS_PAD * CHUNK - N_EDGES
    src2 = jnp.concatenate(
        [edge_index[0].astype(jnp.int32), jnp.zeros((pad,), jnp.int32)]
    ).reshape(N_CHUNKS_PAD, CHUNK)
    dst2 = jnp.concatenate(
        [edge_index[1].astype(jnp.int32), jnp.zeros((pad,), jnp.int32)]
    ).reshape(N_CHUNKS_PAD, CHUNK)
    x_p = jnp.concatenate(
        [x.astype(jnp.int32), jnp.zeros((NP - N_NODES,), jnp.int32)])
    ea3 = edge_attr.reshape(EGRID, 1, EBLK)

    W_AV = jnp.concatenate([W_A, W_V], axis=2)
    b_AV = jnp.concatenate([b_A, b_V], axis=1)
    zeros_np = jnp.zeros((NP, DIM), jnp.float32)

    gather_emb = _make_gather(NP, DIM)
    edge_gather = _make_edge_gather()
    sc_scatter = _sc_scatter_kernel()

    h = gather_emb(atom_emb, x_p)
    e = _edge_init(ea3)

    hav, hb = _node_proj(h, W_AV[0], b_AV[0].reshape(1, 2 * DIM),
                         W_B[0], b_B[0].reshape(1, DIM))
    for i in range(N_LAYERS):
        gav, gb = edge_gather(hav, hb, src2, dst2)
        e, msg, sig = _edge_compute(e, gav, gb, W_C[i],
                                    b_C[i].reshape(1, DIM))
        num, den = sc_scatter(msg, sig, dst2.reshape(-1), zeros_np)
        if i + 1 < N_LAYERS:
            h, hav, hb = _node_update_proj(
                h, num, den, W_U[i], b_U[i].reshape(1, DIM),
                W_AV[i + 1], b_AV[i + 1].reshape(1, 2 * DIM),
                W_B[i + 1], b_B[i + 1].reshape(1, DIM))
        else:
            h = _node_update(h, num, den, W_U[i], b_U[i].reshape(1, DIM))

    out = _out_head(h, W_o1, b_o1.reshape(1, DIM),
                    W_o2[:, 0].reshape(1, DIM), b_o2.reshape(1, 1))
    return out[:N_NODES]


# speedup vs baseline: 3.9392x; 1.0124x over previous
import functools

import jax
import jax.numpy as jnp
from jax import lax
from jax.experimental import pallas as pl
from jax.experimental.pallas import tpu as pltpu
from jax.experimental.pallas import tpu_sc as plsc

DIM = 128
N_LAYERS = 6
CUTOFF = 6.0
N_NODES = 10000
N_EDGES = 320000

NP = 10112
CHUNK = 128
NW = 32
EDGE_CHUNKS = N_EDGES // CHUNK
NODE_CHUNKS = NP // CHUNK
ROWS_PER_TILE = NP // 16

EBLK = 4000
EGRID = (N_EDGES // 2) // EBLK

@functools.cache
def _mesh():
    return plsc.VectorSubcoreMesh(core_axis_name="c", subcore_axis_name="s")


def _sigmoid(v):
    return 1.0 / (1.0 + jnp.exp(-v))


def _edge_init_body(ea_ref, e_ref):
    r = ea_ref[0, 0, :]
    xcol = r[:, None] + 1e-6
    n = (lax.broadcasted_iota(jnp.int32, (1, DIM), 1) + 1).astype(jnp.float32)
    c = CUTOFF
    e_ref[...] = jnp.sqrt(2.0 / c) * jnp.sin(n * (jnp.pi / c) * xcol) / xcol


_edge_init = pl.pallas_call(
    _edge_init_body,
    grid=(EGRID,),
    in_specs=[pl.BlockSpec((1, 1, EBLK), lambda i: (i, 0, 0))],
    out_specs=pl.BlockSpec((EBLK, DIM), lambda i: (i, 0)),
    out_shape=jax.ShapeDtypeStruct((N_EDGES // 2, DIM), jnp.float32),
)


def _node_proj_body(h_ref, wav_ref, bav_ref, wb_ref, bb_ref, hav_ref, hb_ref):
    h = h_ref[...]
    hav_ref[...] = jnp.dot(h, wav_ref[...],
                           preferred_element_type=jnp.float32) + bav_ref[...]
    hb_ref[...] = jnp.dot(h, wb_ref[...],
                          preferred_element_type=jnp.float32) + bb_ref[...]


_node_proj = pl.pallas_call(
    _node_proj_body,
    out_shape=(jax.ShapeDtypeStruct((NP, 2 * DIM), jnp.float32),
               jax.ShapeDtypeStruct((NP, DIM), jnp.float32)),
)


def _edge_compute_body(e_ref, gav_ref, gb_ref, wc_ref, bc_ref,
                       enew_ref, msg_ref, sig_ref):
    e = e_ref[...]
    gav = gav_ref[...]
    m = (gav[:, :DIM] + gb_ref[...]
         + jnp.dot(e, wc_ref[...], preferred_element_type=jnp.float32)
         + bc_ref[...])
    sg = _sigmoid(m)
    sig_ref[...] = sg
    msg_ref[...] = sg * gav[:, DIM:]
    enew_ref[...] = e + m * sg


_edge_compute = pl.pallas_call(
    _edge_compute_body,
    grid=(EGRID,),
    in_specs=[
        pl.BlockSpec((EBLK, DIM), lambda i: (i, 0)),
        pl.BlockSpec((EBLK, 2 * DIM), lambda i: (i, 0)),
        pl.BlockSpec((EBLK, DIM), lambda i: (i, 0)),
        pl.BlockSpec((DIM, DIM), lambda i: (0, 0)),
        pl.BlockSpec((1, DIM), lambda i: (0, 0)),
    ],
    out_specs=[
        pl.BlockSpec((EBLK, DIM), lambda i: (i, 0)),
        pl.BlockSpec((EBLK, DIM), lambda i: (i, 0)),
        pl.BlockSpec((EBLK, DIM), lambda i: (i, 0)),
    ],
    out_shape=(jax.ShapeDtypeStruct((N_EDGES // 2, DIM), jnp.float32),
               jax.ShapeDtypeStruct((N_EDGES // 2, DIM), jnp.float32),
               jax.ShapeDtypeStruct((N_EDGES // 2, DIM), jnp.float32)),
)


def _node_update_body(h_ref, n1_ref, d1_ref, n2_ref, d2_ref,
                      wu_ref, bu_ref, hn_ref):
    h = h_ref[...]
    num = n1_ref[...] + n2_ref[...]
    den = d1_ref[...] + d2_ref[...] + 1e-6
    u = (jnp.dot(h, wu_ref[...], preferred_element_type=jnp.float32)
         + bu_ref[...] + num / den)
    hn_ref[...] = h + u * _sigmoid(u)


_node_update = pl.pallas_call(
    _node_update_body,
    out_shape=jax.ShapeDtypeStruct((NP, DIM), jnp.float32),
)


def _node_update_proj_body(h_ref, n1_ref, d1_ref, n2_ref, d2_ref,
                           wu_ref, bu_ref,
                           wav_ref, bav_ref, wb_ref, bb_ref,
                           hn_ref, hav_ref, hb_ref):
    h = h_ref[...]
    num = n1_ref[...] + n2_ref[...]
    den = d1_ref[...] + d2_ref[...] + 1e-6
    u = (jnp.dot(h, wu_ref[...], preferred_element_type=jnp.float32)
         + bu_ref[...] + num / den)
    hn = h + u * _sigmoid(u)
    hn_ref[...] = hn
    hav_ref[...] = jnp.dot(hn, wav_ref[...],
                           preferred_element_type=jnp.float32) + bav_ref[...]
    hb_ref[...] = jnp.dot(hn, wb_ref[...],
                          preferred_element_type=jnp.float32) + bb_ref[...]


_node_update_proj = pl.pallas_call(
    _node_update_proj_body,
    out_shape=(jax.ShapeDtypeStruct((NP, DIM), jnp.float32),
               jax.ShapeDtypeStruct((NP, 2 * DIM), jnp.float32),
               jax.ShapeDtypeStruct((NP, DIM), jnp.float32)),
)


def _out_head_body(h_ref, w1_ref, b1_ref, w2_ref, b2_ref, o_ref):
    t = (jnp.dot(h_ref[...], w1_ref[...], preferred_element_type=jnp.float32)
         + b1_ref[...])
    t = t * _sigmoid(t)
    z = jnp.sum(t * w2_ref[...], axis=1, keepdims=True) + b2_ref[...]
    o_ref[...] = _sigmoid(z)


_out_head = pl.pallas_call(
    _out_head_body,
    out_shape=jax.ShapeDtypeStruct((NP, 1), jnp.float32),
)


@functools.cache
def _make_gather(n_rows_out, depth):
    n_chunks = n_rows_out // CHUNK
    iters = (n_chunks + NW - 1) // NW

    @functools.partial(
        pl.kernel, mesh=_mesh(),
        out_type=jax.ShapeDtypeStruct((n_rows_out, depth), jnp.float32),
        scratch_types=[
            pltpu.VMEM((CHUNK,), jnp.int32),
            pltpu.VMEM((CHUNK, depth), jnp.float32),
            pltpu.SemaphoreType.DMA,
        ],
    )
    def gather(table_hbm, idx_hbm, out_hbm, idx_v, rows_v, sem):
        w = lax.axis_index("s") * 2 + lax.axis_index("c")

        def body(i, carry):
            g = i * NW + w

            @pl.when(g < n_chunks)
            def _():
                base = g * CHUNK
                pltpu.sync_copy(idx_hbm.at[pl.ds(base, CHUNK)], idx_v)
                pltpu.async_copy(table_hbm.at[idx_v], rows_v, sem).wait()
                pltpu.sync_copy(rows_v, out_hbm.at[pl.ds(base, CHUNK), :])

            return carry

        lax.fori_loop(0, iters, body, 0)

    return gather


E_HALF = N_EDGES // 2
CH_HALF = E_HALF // CHUNK
GATHER_TCH = 40
SCATTER_TCH = 80
N_CHUNKS_PAD = GATHER_TCH * NW


@functools.cache
def _make_edge_gather():
    MAX_CH = GATHER_TCH

    @functools.partial(
        pl.kernel, mesh=_mesh(),
        out_type=(jax.ShapeDtypeStruct((E_HALF, 2 * DIM), jnp.float32),
                  jax.ShapeDtypeStruct((E_HALF, DIM), jnp.float32)),
        scratch_types=[
            pltpu.VMEM((MAX_CH, CHUNK), jnp.int32),
            pltpu.VMEM((MAX_CH, CHUNK), jnp.int32),
            pltpu.VMEM((CHUNK, 2 * DIM), jnp.float32),
            pltpu.VMEM((CHUNK, 2 * DIM), jnp.float32),
            pltpu.VMEM((CHUNK, DIM), jnp.float32),
            pltpu.VMEM((CHUNK, DIM), jnp.float32),
            pltpu.SemaphoreType.DMA,
            pltpu.SemaphoreType.DMA,
            pltpu.SemaphoreType.DMA,
            pltpu.SemaphoreType.DMA,
            pltpu.SemaphoreType.DMA,
        ],
    )
    def gather(tab_av, tab_b, src2, dst2, gav, gb,
               sidx, didx, rav0, rav1, rb0, rb1, lsem, g0, g1, w0, w1):
        w = lax.axis_index("s") * 2 + lax.axis_index("c")
        start = pl.multiple_of(MAX_CH * w, 8)
        nch = jnp.minimum(CH_HALF - MAX_CH * w, MAX_CH)
        RAV, RB, GS, WS = (rav0, rav1), (rb0, rb1), (g0, g1), (w0, w1)

        pltpu.async_copy(src2.at[pl.ds(start, MAX_CH), :], sidx, lsem)
        pltpu.async_copy(dst2.at[pl.ds(start, MAX_CH), :], didx, lsem)
        pltpu.make_async_copy(src2.at[pl.ds(start, MAX_CH), :], sidx,
                              lsem).wait()
        pltpu.make_async_copy(dst2.at[pl.ds(start, MAX_CH), :], didx,
                              lsem).wait()

        def body(i2, carry):
            for u in (0, 1):
                i = i2 * 2 + u
                base = pl.multiple_of((start + i) * CHUNK, CHUNK)
                out_av = gav.at[pl.ds(base, CHUNK), :]
                out_b = gb.at[pl.ds(base, CHUNK), :]

                @pl.when((i >= 2) & (i - 2 < nch))
                def _():
                    pltpu.make_async_copy(RAV[u], out_av, WS[u]).wait()
                    pltpu.make_async_copy(RB[u], out_b, WS[u]).wait()

                @pl.when(i < nch)
                def _():
                    pltpu.async_copy(tab_av.at[sidx.at[i]], RAV[u], GS[u])
                    pltpu.async_copy(tab_b.at[didx.at[i]], RB[u], GS[u])
                    pltpu.make_async_copy(tab_av.at[sidx.at[i]], RAV[u],
                                          GS[u]).wait()
                    pltpu.make_async_copy(tab_b.at[didx.at[i]], RB[u],
                                          GS[u]).wait()
                    pltpu.async_copy(RAV[u], out_av, WS[u])
                    pltpu.async_copy(RB[u], out_b, WS[u])

            return carry

        lax.fori_loop(0, (MAX_CH + 3) // 2, body, 0)

    return gather


@functools.cache
def _sc_scatter_kernel():
    TCH = SCATTER_TCH

    @functools.partial(
        pl.kernel, mesh=_mesh(),
        out_type=(jax.ShapeDtypeStruct((NP, DIM), jnp.float32),
                  jax.ShapeDtypeStruct((NP, DIM), jnp.float32)),
        scratch_types=[
            pltpu.VMEM_SHARED((NP, DIM), jnp.float32),
            pltpu.VMEM((CHUNK,), jnp.int32),
            pltpu.VMEM((CHUNK,), jnp.int32),
            pltpu.VMEM((CHUNK, DIM), jnp.float32),
            pltpu.VMEM((CHUNK, DIM), jnp.float32),
            pltpu.SemaphoreType.DMA,
            pltpu.SemaphoreType.DMA,
            pltpu.SemaphoreType.DMA,
        ],
    )
    def scatter(msg_hbm, sig_hbm, dstf, zeros_hbm,
                num_hbm, den_hbm, accum, i0, i1, r0, r1, l0, l1, ssem):
        c = lax.axis_index("c")
        s = lax.axis_index("s")
        start = pl.multiple_of(s * TCH, 8)
        nch = jnp.minimum(CH_HALF - s * TCH, TCH)
        rows = pl.ds(pl.multiple_of(s * ROWS_PER_TILE, 8), ROWS_PER_TILE)
        R, I, L = (r0, r1), (i0, i1), (l0, l1)

        pltpu.sync_copy(zeros_hbm.at[rows], accum.at[rows])
        plsc.subcore_barrier()

        def issue_loads(src_hbm, i, slot):
            base = pl.multiple_of((start + i) * CHUNK, CHUNK)
            pltpu.async_copy(src_hbm.at[pl.ds(base, CHUNK), :], R[slot],
                             L[slot])
            pltpu.async_copy(dstf.at[pl.ds(base, CHUNK)], I[slot], L[slot])

        def wait_loads(src_hbm, i, slot):
            base = pl.multiple_of((start + i) * CHUNK, CHUNK)
            pltpu.make_async_copy(src_hbm.at[pl.ds(base, CHUNK), :], R[slot],
                                  L[slot]).wait()
            pltpu.make_async_copy(dstf.at[pl.ds(base, CHUNK)], I[slot],
                                  L[slot]).wait()

        def one_phase(src_hbm, out_hbm):
            @pl.when(0 < nch)
            def _():
                issue_loads(src_hbm, 0, 0)

            def body(i2, carry):
                for u in (0, 1):
                    i = i2 * 2 + u

                    @pl.when((i >= 1) & (i - 1 < nch))
                    def _():
                        pltpu.make_async_copy(
                            R[1 - u], accum.at[I[1 - u]], ssem).wait()

                    @pl.when(i + 1 < nch)
                    def _():
                        issue_loads(src_hbm, i + 1, 1 - u)

                    @pl.when(i < nch)
                    def _():
                        wait_loads(src_hbm, i, u)
                        pltpu.async_copy(R[u], accum.at[I[u]], ssem,
                                         add=True)

                return carry

            lax.fori_loop(0, (TCH + 3) // 2, body, 0)
            plsc.subcore_barrier()
            pltpu.sync_copy(accum.at[rows], out_hbm.at[rows])

        @pl.when(c == 0)
        def _():
            one_phase(msg_hbm, num_hbm)

        @pl.when(c == 1)
        def _():
            one_phase(sig_hbm, den_hbm)

    return scatter


def kernel(x, edge_index, edge_attr, atom_emb,
           W_A, b_A, W_B, b_B, W_C, b_C, W_U, b_U, W_V, b_V,
           W_o1, b_o1, W_o2, b_o2):
    pad = N_CHUNKS_PAD * CHUNK - E_HALF
    src_i = edge_index[0].astype(jnp.int32)
    dst_i = edge_index[1].astype(jnp.int32)
    zpad = jnp.zeros((pad,), jnp.int32)
    src2 = [jnp.concatenate([src_i[k * E_HALF:(k + 1) * E_HALF], zpad]
                            ).reshape(N_CHUNKS_PAD, CHUNK) for k in (0, 1)]
    dst2 = [jnp.concatenate([dst_i[k * E_HALF:(k + 1) * E_HALF], zpad]
                            ).reshape(N_CHUNKS_PAD, CHUNK) for k in (0, 1)]
    dstf = [d.reshape(-1) for d in dst2]
    x_p = jnp.concatenate(
        [x.astype(jnp.int32), jnp.zeros((NP - N_NODES,), jnp.int32)])
    ea3 = [edge_attr[k * E_HALF:(k + 1) * E_HALF].reshape(EGRID, 1, EBLK)
           for k in (0, 1)]

    W_AV = jnp.concatenate([W_A, W_V], axis=2)
    b_AV = jnp.concatenate([b_A, b_V], axis=1)
    zeros_np = jnp.zeros((NP, DIM), jnp.float32)

    gather_emb = _make_gather(NP, DIM)
    edge_gather = _make_edge_gather()
    sc_scatter = _sc_scatter_kernel()

    h = gather_emb(atom_emb, x_p)
    e = [_edge_init(ea3[0]), _edge_init(ea3[1])]

    hav, hb = _node_proj(h, W_AV[0], b_AV[0].reshape(1, 2 * DIM),
                         W_B[0], b_B[0].reshape(1, DIM))
    for i in range(N_LAYERS):
        nd = []
        for k in (0, 1):
            gav, gb = edge_gather(hav, hb, src2[k], dst2[k])
            e[k], msg, sig = _edge_compute(e[k], gav, gb, W_C[i],
                                           b_C[i].reshape(1, DIM))
            nd.append(sc_scatter(msg, sig, dstf[k], zeros_np))
        (n1, d1), (n2, d2) = nd
        if i + 1 < N_LAYERS:
            h, hav, hb = _node_update_proj(
                h, n1, d1, n2, d2, W_U[i], b_U[i].reshape(1, DIM),
                W_AV[i + 1], b_AV[i + 1].reshape(1, 2 * DIM),
                W_B[i + 1], b_B[i + 1].reshape(1, DIM))
        else:
            h = _node_update(h, n1, d1, n2, d2, W_U[i],
                             b_U[i].reshape(1, DIM))

    out = _out_head(h, W_o1, b_o1.reshape(1, DIM),
                    W_o2[:, 0].reshape(1, DIM), b_o2.reshape(1, 1))
    return out[:N_NODES]
